# baseline (device time: 111749 ns/iter reference)
import functools

import jax
import jax.numpy as jnp
from jax import lax
from jax.experimental import pallas as pl
from jax.experimental.pallas import tpu as pltpu

N_DEV = 8


def kernel(x, Wq, K_ext, V_ext, Wo):
    B, Sq, D = x.shape
    _, Skv_l, H, Dh = K_ext.shape
    Hl = H // N_DEV
    HDl = Hl * Dh

    Kr = K_ext.reshape(B, Skv_l, H * Dh)
    Vr = V_ext.reshape(B, Skv_l, H * Dh)

    def body(x_ref, wq_ref, k_ref, v_ref, wo_ref, out_ref,
             kv_send, kv_recv, acc_ref, p_recv,
             kv_send_sems, kv_recv_sems, p_send_sems, p_recv_sems,
             local_sem):
        me = lax.axis_index("i")

        barrier_sem = pltpu.get_barrier_semaphore()
        for j in range(N_DEV):
            if_peer = j
            pl.semaphore_signal(
                barrier_sem, inc=1,
                device_id=(if_peer,), device_id_type=pl.DeviceIdType.MESH,
            )
        pl.semaphore_wait(barrier_sem, N_DEV)

        for j in range(N_DEV):
            kv_send[j, 0] = k_ref[:, :, j * HDl:(j + 1) * HDl].astype(jnp.bfloat16)
            kv_send[j, 1] = v_ref[:, :, j * HDl:(j + 1) * HDl].astype(jnp.bfloat16)

        kv_rdmas = []
        for j in range(N_DEV):
            rdma = pltpu.make_async_remote_copy(
                src_ref=kv_send.at[j],
                dst_ref=kv_recv.at[me],
                send_sem=kv_send_sems.at[j],
                recv_sem=kv_recv_sems.at[me],
                device_id=(j,),
                device_id_type=pl.DeviceIdType.MESH,
            )
            kv_rdmas.append(rdma)

            @pl.when(j != me)
            def _(rdma=rdma):
                rdma.start()

        cp = pltpu.make_async_copy(kv_send.at[me], kv_recv.at[me], local_sem)
        cp.start()

        xb = x_ref[...].astype(jnp.bfloat16)
        wq = wq_ref[...].astype(jnp.bfloat16)
        qs = []
        for b in range(B):
            qb = lax.dot_general(
                xb[b], wq, (((1,), (0,)), ((), ())),
                preferred_element_type=jnp.float32,
            ).astype(jnp.bfloat16)
            qs.append(qb)

        cp.wait()
        for s in range(N_DEV):
            recv = pltpu.make_async_remote_copy(
                src_ref=kv_send.at[s],
                dst_ref=kv_recv.at[s],
                send_sem=kv_send_sems.at[s],
                recv_sem=kv_recv_sems.at[s],
                device_id=(s,),
                device_id_type=pl.DeviceIdType.MESH,
            )

            @pl.when(s != me)
            def _(recv=recv):
                recv.wait_recv()

        Skv = N_DEV * Skv_l
        ri = lax.broadcasted_iota(jnp.int32, (Sq, Skv), 0) // 64
        ci = lax.broadcasted_iota(jnp.int32, (Sq, Skv), 1) // 64
        mask = (ri == ci) | (ci == 0) | (((ri + ci) % 3) == 0)

        kvall = kv_recv[...]
        wo = wo_ref[...].astype(jnp.bfloat16)
        for b in range(B):
            ctx_list = []
            for h in range(Hl):
                hs = slice(h * Dh, (h + 1) * Dh)
                kfull = jnp.concatenate(
                    [kvall[s, 0, b, :, hs] for s in range(N_DEV)], axis=0
                )
                vfull = jnp.concatenate(
                    [kvall[s, 1, b, :, hs] for s in range(N_DEV)], axis=0
                )
                q_bh = qs[b][:, hs]
                scores = lax.dot_general(
                    q_bh, kfull, (((1,), (1,)), ((), ())),
                    preferred_element_type=jnp.float32,
                ) * 0.125
                scores = jnp.where(mask, scores, -1e9)
                m = jnp.max(scores, axis=-1, keepdims=True)
                w = jnp.exp(scores - m)
                w = w / jnp.sum(w, axis=-1, keepdims=True)
                ctx = lax.dot_general(
                    w.astype(jnp.bfloat16), vfull, (((1,), (0,)), ((), ())),
                    preferred_element_type=jnp.float32,
                )
                ctx_list.append(ctx)
            ctx_b = jnp.concatenate(ctx_list, axis=1).astype(jnp.bfloat16)
            acc_ref[b] = lax.dot_general(
                ctx_b, wo, (((1,), (0,)), ((), ())),
                preferred_element_type=jnp.float32,
            )

        for j in range(N_DEV):
            rdma = pltpu.make_async_remote_copy(
                src_ref=acc_ref,
                dst_ref=p_recv.at[me],
                send_sem=p_send_sems.at[j],
                recv_sem=p_recv_sems.at[me],
                device_id=(j,),
                device_id_type=pl.DeviceIdType.MESH,
            )

            @pl.when(j != me)
            def _(rdma=rdma):
                rdma.start()

        cp2 = pltpu.make_async_copy(acc_ref, p_recv.at[me], local_sem)
        cp2.start()
        cp2.wait()

        for s in range(N_DEV):
            recv = pltpu.make_async_remote_copy(
                src_ref=acc_ref,
                dst_ref=p_recv.at[s],
                send_sem=p_send_sems.at[s],
                recv_sem=p_recv_sems.at[s],
                device_id=(s,),
                device_id_type=pl.DeviceIdType.MESH,
            )

            @pl.when(s != me)
            def _(recv=recv):
                recv.wait_recv()

        pr = p_recv[...]
        out_ref[...] = jnp.sum(pr, axis=0)

        for j in range(N_DEV):
            @pl.when(j != me)
            def _(j=j):
                kv_rdmas[j].wait_send()
                psend = pltpu.make_async_remote_copy(
                    src_ref=acc_ref,
                    dst_ref=p_recv.at[me],
                    send_sem=p_send_sems.at[j],
                    recv_sem=p_recv_sems.at[me],
                    device_id=(j,),
                    device_id_type=pl.DeviceIdType.MESH,
                )
                psend.wait_send()

    return pl.pallas_call(
        body,
        out_shape=jax.ShapeDtypeStruct((B, Sq, D), jnp.float32),
        in_specs=[pl.BlockSpec(memory_space=pltpu.VMEM)] * 5,
        out_specs=pl.BlockSpec(memory_space=pltpu.VMEM),
        scratch_shapes=[
            pltpu.VMEM((N_DEV, 2, B, Skv_l, HDl), jnp.bfloat16),
            pltpu.VMEM((N_DEV, 2, B, Skv_l, HDl), jnp.bfloat16),
            pltpu.VMEM((B, Sq, D), jnp.float32),
            pltpu.VMEM((N_DEV, B, Sq, D), jnp.float32),
            pltpu.SemaphoreType.DMA((N_DEV,)),
            pltpu.SemaphoreType.DMA((N_DEV,)),
            pltpu.SemaphoreType.DMA((N_DEV,)),
            pltpu.SemaphoreType.DMA((N_DEV,)),
            pltpu.SemaphoreType.DMA,
        ],
        compiler_params=pltpu.CompilerParams(collective_id=0),
    )(x, Wq, Kr, Vr, Wo)


# device time: 66535 ns/iter; 1.6796x vs baseline; 1.6796x over previous
import jax
import jax.numpy as jnp
from jax import lax
from jax.experimental import pallas as pl
from jax.experimental.pallas import tpu as pltpu

N_DEV = 8


def kernel(x, Wq, K_ext, V_ext, Wo):
    B, Sq, D = x.shape
    _, Skv_l, H, Dh = K_ext.shape
    Hl = H // N_DEV
    HDl = Hl * Dh
    Sq_sl = Sq // N_DEV

    Kr = K_ext.reshape(B, Skv_l, H * Dh)
    Vr = V_ext.reshape(B, Skv_l, H * Dh)

    def body(x_ref, wq_ref, k_ref, v_ref, wo_ref, out_ref,
             kv_send, kv_recv, acc_ref,
             rs_send, rs_recv, ag_send, ag_recv,
             kv_send_sems, kv_recv_sems,
             rs_send_sems, rs_recv_sems, ag_send_sems, ag_recv_sems,
             local_sem):
        me = lax.axis_index("i")

        for j in range(N_DEV):
            kv_send[j, 0] = k_ref[:, :, j * HDl:(j + 1) * HDl].astype(jnp.bfloat16)
            kv_send[j, 1] = v_ref[:, :, j * HDl:(j + 1) * HDl].astype(jnp.bfloat16)

        barrier_sem = pltpu.get_barrier_semaphore()
        for j in range(N_DEV):
            pl.semaphore_signal(
                barrier_sem, inc=1,
                device_id=(j,), device_id_type=pl.DeviceIdType.MESH,
            )
        pl.semaphore_wait(barrier_sem, N_DEV)

        kv_rdmas = []
        for j in range(N_DEV):
            rdma = pltpu.make_async_remote_copy(
                src_ref=kv_send.at[j],
                dst_ref=kv_recv.at[me],
                send_sem=kv_send_sems.at[j],
                recv_sem=kv_recv_sems.at[me],
                device_id=(j,),
                device_id_type=pl.DeviceIdType.MESH,
            )
            kv_rdmas.append(rdma)

            @pl.when(j != me)
            def _(rdma=rdma):
                rdma.start()

        cp = pltpu.make_async_copy(kv_send.at[me], kv_recv.at[me], local_sem)
        cp.start()

        xb = x_ref[...].astype(jnp.bfloat16)
        wq = wq_ref[...].astype(jnp.bfloat16)
        qs = []
        for b in range(B):
            qb = lax.dot_general(
                xb[b], wq, (((1,), (0,)), ((), ())),
                preferred_element_type=jnp.float32,
            ).astype(jnp.bfloat16)
            qs.append(qb)

        cp.wait()
        for s in range(N_DEV):
            recv = pltpu.make_async_remote_copy(
                src_ref=kv_send.at[s],
                dst_ref=kv_recv.at[s],
                send_sem=kv_send_sems.at[s],
                recv_sem=kv_recv_sems.at[s],
                device_id=(s,),
                device_id_type=pl.DeviceIdType.MESH,
            )

            @pl.when(s != me)
            def _(recv=recv):
                recv.wait_recv()

        Skv = N_DEV * Skv_l
        ri = lax.broadcasted_iota(jnp.int32, (Sq, Skv), 0) // 64
        ci = lax.broadcasted_iota(jnp.int32, (Sq, Skv), 1) // 64
        mask = (ri == ci) | (ci == 0) | (((ri + ci) % 3) == 0)

        kvall = kv_recv[...]
        wo = wo_ref[...].astype(jnp.bfloat16)
        for b in range(B):
            ctx_list = []
            for h in range(Hl):
                hs = slice(h * Dh, (h + 1) * Dh)
                kfull = jnp.concatenate(
                    [kvall[s, 0, b, :, hs] for s in range(N_DEV)], axis=0
                )
                vfull = jnp.concatenate(
                    [kvall[s, 1, b, :, hs] for s in range(N_DEV)], axis=0
                )
                q_bh = qs[b][:, hs]
                scores = lax.dot_general(
                    q_bh, kfull, (((1,), (1,)), ((), ())),
                    preferred_element_type=jnp.float32,
                ) * 0.125
                scores = jnp.where(mask, scores, -1e9)
                m = jnp.max(scores, axis=-1, keepdims=True)
                w = jnp.exp(scores - m)
                w = w / jnp.sum(w, axis=-1, keepdims=True)
                ctx = lax.dot_general(
                    w.astype(jnp.bfloat16), vfull, (((1,), (0,)), ((), ())),
                    preferred_element_type=jnp.float32,
                )
                ctx_list.append(ctx)
            ctx_b = jnp.concatenate(ctx_list, axis=1).astype(jnp.bfloat16)
            acc_ref[b] = lax.dot_general(
                ctx_b, wo, (((1,), (0,)), ((), ())),
                preferred_element_type=jnp.float32,
            )

        accv = acc_ref[...]
        for j in range(N_DEV):
            rs_send[j] = accv[:, j * Sq_sl:(j + 1) * Sq_sl, :].astype(jnp.bfloat16)

        rs_rdmas = []
        for j in range(N_DEV):
            rdma = pltpu.make_async_remote_copy(
                src_ref=rs_send.at[j],
                dst_ref=rs_recv.at[me],
                send_sem=rs_send_sems.at[j],
                recv_sem=rs_recv_sems.at[me],
                device_id=(j,),
                device_id_type=pl.DeviceIdType.MESH,
            )
            rs_rdmas.append(rdma)

            @pl.when(j != me)
            def _(rdma=rdma):
                rdma.start()

        cp2 = pltpu.make_async_copy(rs_send.at[me], rs_recv.at[me], local_sem)
        cp2.start()
        cp2.wait()

        for s in range(N_DEV):
            recv = pltpu.make_async_remote_copy(
                src_ref=rs_send.at[s],
                dst_ref=rs_recv.at[s],
                send_sem=rs_send_sems.at[s],
                recv_sem=rs_recv_sems.at[s],
                device_id=(s,),
                device_id_type=pl.DeviceIdType.MESH,
            )

            @pl.when(s != me)
            def _(recv=recv):
                recv.wait_recv()

        rsv = rs_recv[...]
        red = jnp.zeros((B, Sq_sl, D), jnp.float32)
        for s in range(N_DEV):
            red = red + rsv[s].astype(jnp.float32)
        ag_send[...] = red.astype(jnp.bfloat16)

        ag_rdmas = []
        for j in range(N_DEV):
            rdma = pltpu.make_async_remote_copy(
                src_ref=ag_send,
                dst_ref=ag_recv.at[me],
                send_sem=ag_send_sems.at[j],
                recv_sem=ag_recv_sems.at[me],
                device_id=(j,),
                device_id_type=pl.DeviceIdType.MESH,
            )
            ag_rdmas.append(rdma)

            @pl.when(j != me)
            def _(rdma=rdma):
                rdma.start()

        cp3 = pltpu.make_async_copy(ag_send, ag_recv.at[me], local_sem)
        cp3.start()
        cp3.wait()

        for s in range(N_DEV):
            recv = pltpu.make_async_remote_copy(
                src_ref=ag_send,
                dst_ref=ag_recv.at[s],
                send_sem=ag_send_sems.at[s],
                recv_sem=ag_recv_sems.at[s],
                device_id=(s,),
                device_id_type=pl.DeviceIdType.MESH,
            )

            @pl.when(s != me)
            def _(recv=recv):
                recv.wait_recv()

        agv = ag_recv[...]
        for s in range(N_DEV):
            out_ref[:, s * Sq_sl:(s + 1) * Sq_sl, :] = agv[s].astype(jnp.float32)

        for j in range(N_DEV):
            @pl.when(j != me)
            def _(j=j):
                kv_rdmas[j].wait_send()
                rs_rdmas[j].wait_send()
                ag_rdmas[j].wait_send()

    return pl.pallas_call(
        body,
        out_shape=jax.ShapeDtypeStruct((B, Sq, D), jnp.float32),
        in_specs=[pl.BlockSpec(memory_space=pltpu.VMEM)] * 5,
        out_specs=pl.BlockSpec(memory_space=pltpu.VMEM),
        scratch_shapes=[
            pltpu.VMEM((N_DEV, 2, B, Skv_l, HDl), jnp.bfloat16),
            pltpu.VMEM((N_DEV, 2, B, Skv_l, HDl), jnp.bfloat16),
            pltpu.VMEM((B, Sq, D), jnp.float32),
            pltpu.VMEM((N_DEV, B, Sq_sl, D), jnp.bfloat16),
            pltpu.VMEM((N_DEV, B, Sq_sl, D), jnp.bfloat16),
            pltpu.VMEM((B, Sq_sl, D), jnp.bfloat16),
            pltpu.VMEM((N_DEV, B, Sq_sl, D), jnp.bfloat16),
            pltpu.SemaphoreType.DMA((N_DEV,)),
            pltpu.SemaphoreType.DMA((N_DEV,)),
            pltpu.SemaphoreType.DMA((N_DEV,)),
            pltpu.SemaphoreType.DMA((N_DEV,)),
            pltpu.SemaphoreType.DMA((N_DEV,)),
            pltpu.SemaphoreType.DMA((N_DEV,)),
            pltpu.SemaphoreType.DMA,
        ],
        compiler_params=pltpu.CompilerParams(collective_id=0),
    )(x, Wq, Kr, Vr, Wo)
